# baseline (speedup 1.0000x reference)
import functools

import jax
import jax.numpy as jnp
import numpy as np
from jax import lax
from jax.experimental import pallas as pl
from jax.experimental.pallas import tpu as pltpu
from jax.experimental.pallas import tpu_sc as plsc

NEG = -1e30


def _mlp(layers, x, plain_last=True):
    shp = x.shape
    x = x.reshape(-1, shp[-1])
    n = len(layers)
    for i, lyr in enumerate(layers):
        x = x @ lyr["W"] + lyr["b"]
        last = i == n - 1
        if (not last) or (not plain_last):
            if lyr["g"] is not None:
                mu = jnp.mean(x, axis=0, keepdims=True)
                var = jnp.var(x, axis=0, keepdims=True)
                x = (x - mu) / jnp.sqrt(var + 1e-5) * lyr["g"] + lyr["bb"]
            x = jax.nn.relu(x)
    return x.reshape(shp[:-1] + (x.shape[-1],))


def _fps_body(px_ref, py_ref, pz_ref, idx_ref, carry_ref):
    N, B = px_ref.shape
    m = idx_ref.shape[0]
    px = px_ref[...]
    py = py_ref[...]
    pz = pz_ref[...]
    d0 = (
        (px - px[0:1, :]) ** 2
        + (py - py[0:1, :]) ** 2
        + (pz - pz[0:1, :]) ** 2
    )
    carry_ref[...] = d0
    idx_ref[0:1, :] = jnp.zeros((1, B), jnp.int32)
    iota = lax.broadcasted_iota(jnp.int32, (N, B), 0)

    def step(t, _):
        carry = carry_ref[...]
        maxv = jnp.max(carry, axis=0, keepdims=True)
        cand = jnp.where(carry == maxv, iota, N)
        nxt = jnp.min(cand, axis=0, keepdims=True)
        oh = iota == nxt
        xn = jnp.sum(jnp.where(oh, px, 0.0), axis=0, keepdims=True)
        yn = jnp.sum(jnp.where(oh, py, 0.0), axis=0, keepdims=True)
        zn = jnp.sum(jnp.where(oh, pz, 0.0), axis=0, keepdims=True)
        d2 = (px - xn) ** 2 + (py - yn) ** 2 + (pz - zn) ** 2
        carry_ref[...] = jnp.minimum(carry, d2)
        idx_ref[pl.ds(t, 1), :] = nxt.astype(jnp.int32)
        return 0

    lax.fori_loop(1, m, step, 0)


def _fps_batched(pos, m):
    B, N, _ = pos.shape
    px = pos[..., 0].T
    py = pos[..., 1].T
    pz = pos[..., 2].T
    idx_t = pl.pallas_call(
        _fps_body,
        out_shape=jax.ShapeDtypeStruct((m, B), jnp.int32),
        scratch_shapes=[pltpu.VMEM((N, B), jnp.float32)],
    )(px, py, pz)
    return idx_t.T


def _sc_gather_rows(table, gidx):
    V, C = table.shape
    (M,) = gidx.shape
    info = plsc.get_sparse_core_info()
    NC, NS = info.num_cores, info.num_subcores
    NW = NC * NS
    CH = 128 if C <= 384 else 64
    rpw = M // NW
    nch = rpw // CH
    assert rpw % CH == 0, (M, C, CH)
    mesh = plsc.VectorSubcoreMesh(core_axis_name="c", subcore_axis_name="s")

    @functools.partial(
        pl.kernel,
        mesh=mesh,
        out_type=jax.ShapeDtypeStruct((M, C), jnp.float32),
        scratch_types=[
            pltpu.VMEM((CH,), jnp.int32),
            pltpu.VMEM((CH, C), jnp.float32),
            pltpu.SemaphoreType.DMA,
        ],
    )
    def k(table_hbm, idx_hbm, out_hbm, idx_v, rows_v, sem):
        wid = lax.axis_index("s") * NC + lax.axis_index("c")
        base = wid * rpw

        def body(j, _):
            off = base + j * CH
            pltpu.sync_copy(idx_hbm.at[pl.ds(off, CH)], idx_v)
            pltpu.async_copy(table_hbm.at[idx_v], rows_v, sem).wait()
            pltpu.sync_copy(rows_v, out_hbm.at[pl.ds(off, CH)])
            return 0

        lax.fori_loop(0, nch, body, 0, unroll=False)

    return k(table, gidx)


def _gather_nodes_sc(x, nbr):
    B, N, C = x.shape
    _, m, k = nbr.shape
    Cp = ((C + 127) // 128) * 128
    CH = 128 if Cp <= 384 else 64
    if Cp > C:
        x = jnp.concatenate(
            [x, jnp.zeros((B, N, Cp - C), jnp.float32)], axis=-1
        )
    table = x.reshape(B * N, Cp)
    gidx = (nbr + (jnp.arange(B, dtype=jnp.int32) * N)[:, None, None]).reshape(-1)
    M = gidx.shape[0]
    Mp = ((M + 32 * CH - 1) // (32 * CH)) * (32 * CH)
    if Mp > M:
        gidx = jnp.concatenate([gidx, jnp.zeros((Mp - M,), jnp.int32)])
    out = _sc_gather_rows(table, gidx)
    return out[:M].reshape(B, m, k, Cp)


def _sa(x, pos, ratio, r, layers):
    B, N, _ = pos.shape
    m = int(N * ratio)
    idx = _fps_batched(pos, m)
    pos_dst = jnp.take_along_axis(pos, idx[..., None], axis=1)
    d2 = jnp.sum((pos_dst[:, :, None, :] - pos[:, None, :, :]) ** 2, axis=-1)
    k = min(64, N)
    neg, nbr = lax.top_k(-d2, k)
    mask = (-neg) <= r * r
    C = x.shape[-1]
    g = _gather_nodes_sc(jnp.concatenate([x, pos], axis=-1), nbr)
    x_j = g[..., :C]
    pos_j = g[..., C : C + 3]
    msg = jnp.concatenate([x_j, pos_j - pos_dst[:, :, None, :]], axis=-1)
    h = _mlp(layers, msg)
    h = jnp.where(mask[..., None], h, NEG)
    out = jnp.max(h, axis=2)
    out = jnp.where(jnp.any(mask, axis=2)[..., None], out, 0.0)
    return out, pos_dst


def _td(x, pos, ratio, kk, layers):
    B, N, _ = x.shape
    m = int(N * ratio)
    idx = _fps_batched(pos, m)
    pos_dst = jnp.take_along_axis(pos, idx[..., None], axis=1)
    d2 = jnp.sum((pos_dst[:, :, None, :] - pos[:, None, :, :]) ** 2, axis=-1)
    _, nbr = lax.top_k(-d2, kk)
    xf = _mlp(layers, x, plain_last=False)
    xg = _gather_nodes_sc(xf, nbr)[..., : xf.shape[-1]]
    return jnp.max(xg, axis=2), pos_dst


def _identity_pallas(y):
    def body(x_ref, o_ref):
        o_ref[...] = x_ref[...]

    return pl.pallas_call(
        body, out_shape=jax.ShapeDtypeStruct(y.shape, y.dtype)
    )(y)


def kernel(data, params):
    betas = jnp.linspace(1e-4, 0.02, 1000)
    t = jax.random.randint(jax.random.key(1), (), 0, 1000)
    noise = jax.random.normal(jax.random.key(2), data.shape, jnp.float32)
    bt = betas[t]
    noisy = jnp.sqrt(1.0 - bt) * data + jnp.sqrt(bt) * noise
    x1, p1 = _sa(noisy, noisy, 0.5, 0.2, params["sa1"])
    x1d, p1d = _td(x1, p1, 0.25, 16, params["td1"])
    x2, p2 = _sa(x1d, p1d, 0.25, 0.4, params["sa2"])
    x2d, p2d = _td(x2, p2, 0.25, 16, params["td2"])
    h = _mlp(params["sa3"], jnp.concatenate([x2d, p2d], axis=-1))
    g = jnp.mean(h, axis=1)
    den = _mlp(params["rev"], g)
    y = _mlp(params["cls"], den)
    return _identity_pallas(y)
